# baseline (speedup 1.0000x reference)
import functools

import jax
import jax.numpy as jnp
from jax import lax
from jax.experimental import pallas as pl
from jax.experimental.pallas import tpu as pltpu
from jax.experimental.pallas import tpu_sc as plsc

N = 256
FEAT = 128
LATENT = 256
NF = 5
BI = 8
GRID = N // BI

SC_NC = 2
SC_NS = 16
NW = SC_NC * SC_NS
BPW = N // NW


@functools.partial(
    pl.kernel,
    mesh=plsc.VectorSubcoreMesh(core_axis_name="c", subcore_axis_name="s"),
    out_type=jax.ShapeDtypeStruct((NF, N, FEAT), jnp.float32),
    scratch_types=[
        pltpu.VMEM((BPW,), jnp.int32),
        pltpu.VMEM((BPW, FEAT), jnp.float32),
        pltpu.SemaphoreType.DMA,
    ],
)
def _sc_gather(t0, i0, t1, i1, t2, i2, t3, i3, t4, i4, out_hbm,
               idx_v, rows_v, sem):
    wid = lax.axis_index("s") * SC_NC + lax.axis_index("c")
    base = wid * BPW
    for f, (tab, ids) in enumerate(
            ((t0, i0), (t1, i1), (t2, i2), (t3, i3), (t4, i4))):
        pltpu.sync_copy(ids.at[pl.ds(base, BPW)], idx_v)
        pltpu.async_copy(tab.at[idx_v], rows_v, sem).wait()
        pltpu.sync_copy(rows_v, out_hbm.at[f, pl.ds(base, BPW)])

_DN = (((1,), (1,)), ((), ()))


def _tc_body(embs_ref, W_ref, b_ref, o0, o1, o2, o3, o4, p1_ref, p2_ref):
    ib = pl.program_id(0)

    @pl.when(ib == 0)
    def _():
        for f in range(NF):
            E = embs_ref[f]
            W = W_ref[f]
            p1_ref[f] = lax.dot_general(E, W[:, :FEAT], _DN,
                                        preferred_element_type=jnp.float32)
            p2_ref[f] = (lax.dot_general(E, W[:, FEAT:2 * FEAT], _DN,
                                         preferred_element_type=jnp.float32)
                         + b_ref[f])

    start = ib * BI
    outs = (o0, o1, o2, o3, o4)
    for f in range(NF):
        E = embs_ref[f]
        e_blk = embs_ref[f, pl.ds(start, BI), :]
        R = e_blk[:, None, :] * E[None, :, :]
        M = lax.dot_general(R, W_ref[f][:, 2 * FEAT:],
                            (((2,), (1,)), ((), ())),
                            preferred_element_type=jnp.float32)
        p1_blk = p1_ref[f, pl.ds(start, BI), :]
        out3 = jnp.maximum(M + p1_blk[:, None, :] + p2_ref[f][None, :, :], 0.0)
        outs[f][...] = out3.reshape(BI * N, LATENT)


@functools.partial(jax.jit, static_argnames=("interpret",))
def _encode(embs, W, b, interpret=False):
    return pl.pallas_call(
        _tc_body,
        grid=(GRID,),
        in_specs=[
            pl.BlockSpec((NF, N, FEAT), lambda i: (0, 0, 0)),
            pl.BlockSpec((NF, LATENT, 3 * FEAT), lambda i: (0, 0, 0)),
            pl.BlockSpec((NF, 1, LATENT), lambda i: (0, 0, 0)),
        ],
        out_specs=[pl.BlockSpec((BI * N, LATENT), lambda i: (i, 0))] * NF,
        out_shape=[jax.ShapeDtypeStruct((N * N, LATENT), jnp.float32)] * NF,
        scratch_shapes=[pltpu.VMEM((NF, N, LATENT), jnp.float32)] * 2,
        interpret=interpret,
    )(embs, W, b)


def kernel(typ_ids, typ_table, typ_W, typ_b, pol_ids, pol_table, pol_W, pol_b,
           mod_ids, mod_table, mod_W, mod_b, gen_ids, gen_table, gen_W, gen_b,
           ten_ids, ten_table, ten_W, ten_b):
    ids = tuple(i.astype(jnp.int32)
                for i in (typ_ids, pol_ids, mod_ids, gen_ids, ten_ids))
    tables = (typ_table, pol_table, mod_table, gen_table, ten_table)
    embs = _sc_gather(tables[0], ids[0], tables[1], ids[1], tables[2], ids[2],
                      tables[3], ids[3], tables[4], ids[4])
    W = jnp.stack((typ_W, pol_W, mod_W, gen_W, ten_W))
    b = jnp.stack((typ_b, pol_b, mod_b, gen_b, ten_b)).reshape(NF, 1, LATENT)
    return tuple(_encode(embs, W, b))


# speedup vs baseline: 1.0371x; 1.0371x over previous
import functools

import jax
import jax.numpy as jnp
from jax import lax
from jax.experimental import pallas as pl
from jax.experimental.pallas import tpu as pltpu
from jax.experimental.pallas import tpu_sc as plsc

N = 256
FEAT = 128
LATENT = 256
NF = 5
BI = 8
GRID = N // BI

SC_NC = 2
SC_NS = 16
NW = SC_NC * SC_NS
BPW = N // NW


@functools.partial(
    pl.kernel,
    mesh=plsc.VectorSubcoreMesh(core_axis_name="c", subcore_axis_name="s"),
    out_type=jax.ShapeDtypeStruct((NF, N, FEAT), jnp.float32),
    scratch_types=(
        [pltpu.VMEM((BPW,), jnp.int32)] * NF
        + [pltpu.VMEM((BPW, FEAT), jnp.float32)] * NF
        + [pltpu.SemaphoreType.DMA]
    ),
)
def _sc_gather(t0, i0, t1, i1, t2, i2, t3, i3, t4, i4, out_hbm,
               x0, x1, x2, x3, x4, r0, r1, r2, r3, r4, sem):
    wid = lax.axis_index("s") * SC_NC + lax.axis_index("c")
    base = wid * BPW
    tabs = (t0, t1, t2, t3, t4)
    ids = (i0, i1, i2, i3, i4)
    idx = (x0, x1, x2, x3, x4)
    rows = (r0, r1, r2, r3, r4)
    for c in [pltpu.async_copy(ids[f].at[pl.ds(base, BPW)], idx[f], sem)
              for f in range(NF)]:
        c.wait()
    for c in [pltpu.async_copy(tabs[f].at[idx[f]], rows[f], sem)
              for f in range(NF)]:
        c.wait()
    for c in [pltpu.async_copy(rows[f], out_hbm.at[f, pl.ds(base, BPW)], sem)
              for f in range(NF)]:
        c.wait()

_DN = (((1,), (1,)), ((), ()))


def _tc_body(embs_ref, W_ref, b_ref, o0, o1, o2, o3, o4, p1_ref, p2_ref):
    ib = pl.program_id(0)

    @pl.when(ib == 0)
    def _():
        for f in range(NF):
            E = embs_ref[f]
            W = W_ref[f]
            p1_ref[f] = lax.dot_general(E, W[:, :FEAT], _DN,
                                        preferred_element_type=jnp.float32)
            p2_ref[f] = (lax.dot_general(E, W[:, FEAT:2 * FEAT], _DN,
                                         preferred_element_type=jnp.float32)
                         + b_ref[f])

    start = ib * BI
    outs = (o0, o1, o2, o3, o4)
    for f in range(NF):
        E = embs_ref[f]
        e_blk = embs_ref[f, pl.ds(start, BI), :]
        R = e_blk[:, None, :] * E[None, :, :]
        M = lax.dot_general(R, W_ref[f][:, 2 * FEAT:],
                            (((2,), (1,)), ((), ())),
                            preferred_element_type=jnp.float32)
        p1_blk = p1_ref[f, pl.ds(start, BI), :]
        out3 = jnp.maximum(M + p1_blk[:, None, :] + p2_ref[f][None, :, :], 0.0)
        outs[f][...] = out3.reshape(BI * N, LATENT)


@functools.partial(jax.jit, static_argnames=("interpret",))
def _encode(embs, W, b, interpret=False):
    return pl.pallas_call(
        _tc_body,
        grid=(GRID,),
        in_specs=[
            pl.BlockSpec((NF, N, FEAT), lambda i: (0, 0, 0)),
            pl.BlockSpec((NF, LATENT, 3 * FEAT), lambda i: (0, 0, 0)),
            pl.BlockSpec((NF, 1, LATENT), lambda i: (0, 0, 0)),
        ],
        out_specs=[pl.BlockSpec((BI * N, LATENT), lambda i: (i, 0))] * NF,
        out_shape=[jax.ShapeDtypeStruct((N * N, LATENT), jnp.float32)] * NF,
        scratch_shapes=[pltpu.VMEM((NF, N, LATENT), jnp.float32)] * 2,
        interpret=interpret,
    )(embs, W, b)


def kernel(typ_ids, typ_table, typ_W, typ_b, pol_ids, pol_table, pol_W, pol_b,
           mod_ids, mod_table, mod_W, mod_b, gen_ids, gen_table, gen_W, gen_b,
           ten_ids, ten_table, ten_W, ten_b):
    ids = tuple(i.astype(jnp.int32)
                for i in (typ_ids, pol_ids, mod_ids, gen_ids, ten_ids))
    embs = _sc_gather(typ_table, ids[0], pol_table, ids[1], mod_table, ids[2],
                      gen_table, ids[3], ten_table, ids[4])
    W = jnp.stack((typ_W, pol_W, mod_W, gen_W, ten_W))
    b = jnp.stack((typ_b, pol_b, mod_b, gen_b, ten_b)).reshape(NF, 1, LATENT)
    return tuple(_encode(embs, W, b))
